# Initial kernel scaffold; baseline (speedup 1.0000x reference)
import jax
import jax.numpy as jnp
from jax.experimental import pallas as pl


def kernel(node_features, node_idx, hedge_idx, W1, b1, W2, b2, W3, b3, W_out, b_out):
    raise NotImplementedError("write your pallas kernel here")


# speedup vs baseline: 1.0013x; 1.0013x over previous
import jax
import jax.numpy as jnp
from jax.experimental import pallas as pl

N = 10000
E = 5000
D_H = 64
L0 = 1.0
L1 = 1.0
ALPHA = 0.1
ITERS = 10


def _head_body(x_ref, w_ref, b_ref, o_ref):
    o_ref[...] = jnp.dot(x_ref[...], w_ref[...],
                         preferred_element_type=jnp.float32) + b_ref[...]


def _layer(X, W, b, node_idx, hedge_idx):
    f = X @ W + b
    ones = jnp.ones((node_idx.shape[0],), dtype=jnp.float32)
    De = jax.ops.segment_sum(ones, hedge_idx, num_segments=E)
    De = jnp.maximum(De, 1.0)
    D0 = jax.ops.segment_sum(De[hedge_idx], node_idx, num_segments=N)
    D1 = jax.ops.segment_sum(ones, node_idx, num_segments=N)
    Dt = L0 * D0 + L1 * D1 + 1.0
    coeff = (L0 + L1 / De)[:, None]
    Y = f
    for _ in range(ITERS):
        Ye = jax.ops.segment_sum(Y[node_idx], hedge_idx, num_segments=E)
        msg = jax.ops.segment_sum((coeff * Ye)[hedge_idx], node_idx, num_segments=N)
        Y = jax.nn.relu((1.0 - ALPHA) * Y + ALPHA * (msg + f) / Dt[:, None])
    return Y


def kernel(node_features, node_idx, hedge_idx, W1, b1, W2, b2, W3, b3, W_out, b_out):
    x = node_features
    for i, (W, b) in enumerate([(W1, b1), (W2, b2), (W3, b3)]):
        residual = x if x.shape[-1] == D_H else None
        y = _layer(x, W, b, node_idx, hedge_idx)
        y = jax.nn.relu(y)
        x = y + residual if residual is not None else y
    return pl.pallas_call(
        _head_body,
        out_shape=jax.ShapeDtypeStruct((N, W_out.shape[1]), jnp.float32),
    )(x, W_out, b_out)


# speedup vs baseline: 1.3848x; 1.3830x over previous
import functools

import jax
import jax.numpy as jnp
from jax import lax
from jax.experimental import pallas as pl
from jax.experimental.pallas import tpu as pltpu
from jax.experimental.pallas import tpu_sc as plsc

N = 10000
E = 5000
D_H = 64
L0 = 1.0
L1 = 1.0
ALPHA = 0.1
ITERS = 10

NW = 16
EPW = 320
EPAD = NW * EPW
NPW = 640
NPAD = NW * NPW
G = 256
U = 128
NSL = D_H // 16
CPAD = 336
IPAD = 656


def _sc_layer_body(f_hbm, nA_hbm, hA_hbm, hB_hbm, nB_hbm, starts_hbm,
                   coeff_hbm, invdt_hbm,
                   y_hbm, yec_hbm,
                   srcb, tgtb, rowbuf, acc, coeffl, invdtl, yloc, fbuf,
                   stab, sem):
    t = lax.axis_index("s")
    e0 = t * EPW
    n0 = t * NPW

    pltpu.sync_copy(starts_hbm, stab)
    pltpu.sync_copy(coeff_hbm.at[t], coeffl)
    pltpu.sync_copy(invdt_hbm.at[t], invdtl)

    pltpu.sync_copy(f_hbm.at[pl.ds(n0, NPW)], yloc)
    pltpu.sync_copy(yloc, y_hbm.at[pl.ds(n0, NPW)])
    plsc.subcore_barrier()

    zvec = jnp.zeros((16,), jnp.float32)

    def zero_acc(nrows):
        def zb(r, c):
            for j in range(NSL):
                acc[r, pl.ds(16 * j, 16)] = zvec
            return c
        lax.fori_loop(0, nrows, zb, 0)

    def gather_accum(idx_hbm, tgt_hbm, table_hbm, sbase, cnt, base):
        a = (sbase // 8) * 8
        tot = (sbase - a) + cnt
        nch = (tot + G - 1) // G

        def chunk(c, carry):
            off = a + c * G
            pltpu.sync_copy(idx_hbm.at[pl.ds(off, G)], srcb)
            pltpu.sync_copy(tgt_hbm.at[pl.ds(off, G)], tgtb.at[pl.ds(0, G)])
            pltpu.async_copy(table_hbm.at[srcb], rowbuf, sem).wait()
            lo = jnp.maximum(sbase - off, 0)
            hi = jnp.minimum(sbase + cnt - off, G)

            def entry(k, cc):
                tgt = tgtb[pl.ds(k, 16)][0] - base
                for j in range(NSL):
                    plsc.addupdate(acc.at[tgt, pl.ds(16 * j, 16)],
                                   rowbuf[k, pl.ds(16 * j, 16)])
                return cc
            lax.fori_loop(lo, hi, entry, 0)
            return carry
        lax.fori_loop(0, nch, chunk, 0)

    def iteration(i, carry):
        zero_acc(EPW)
        sv = stab[0, pl.ds(t, 16)]
        sA = sv[0]
        cA = sv[1] - sA
        gather_accum(nA_hbm, hA_hbm, y_hbm, sA, cA, e0)

        def scale(r, c):
            cf = coeffl[pl.ds(r, 16)][0]
            for j in range(NSL):
                acc[r, pl.ds(16 * j, 16)] = acc[r, pl.ds(16 * j, 16)] * cf
            return c
        lax.fori_loop(0, EPW, scale, 0)
        pltpu.sync_copy(acc.at[pl.ds(0, EPW)], yec_hbm.at[pl.ds(e0, EPW)])
        plsc.subcore_barrier()

        zero_acc(NPW)
        sw = stab[1, pl.ds(t, 16)]
        sB = sw[0]
        cB = sw[1] - sB
        gather_accum(hB_hbm, nB_hbm, yec_hbm, sB, cB, n0)

        def upd_chunk(uc, c):
            pltpu.sync_copy(f_hbm.at[pl.ds(n0 + uc * U, U)], fbuf)

            def upd(r, cc):
                rr = uc * U + r
                inv = invdtl[pl.ds(rr, 16)][0]
                for j in range(NSL):
                    sl = pl.ds(16 * j, 16)
                    v = ((1.0 - ALPHA) * yloc[rr, sl]
                         + (ALPHA * inv) * (acc[rr, sl] + fbuf[r, sl]))
                    yloc[rr, sl] = jnp.maximum(v, 0.0)
                return cc
            lax.fori_loop(0, U, upd, 0)
            return c
        lax.fori_loop(0, NPW // U, upd_chunk, 0)
        pltpu.sync_copy(yloc, y_hbm.at[pl.ds(n0, NPW)])
        plsc.subcore_barrier()
        return carry

    lax.fori_loop(0, ITERS, iteration, 0)


@jax.jit
def _sc_layer(f, nA, hA, hB, nB, starts, coeff_pad, invdt_pad):
    mesh = plsc.VectorSubcoreMesh(core_axis_name="c", subcore_axis_name="s",
                                  num_cores=1)
    y, _ = pl.kernel(
        _sc_layer_body,
        out_type=(jax.ShapeDtypeStruct((NPAD, D_H), jnp.float32),
                  jax.ShapeDtypeStruct((EPAD, D_H), jnp.float32)),
        mesh=mesh,
        compiler_params=pltpu.CompilerParams(use_tc_tiling_on_sc=False),
        scratch_types=[
            pltpu.VMEM((G,), jnp.int32),
            pltpu.VMEM((G + 16,), jnp.int32),
            pltpu.VMEM((G, D_H), jnp.float32),
            pltpu.VMEM((NPW, D_H), jnp.float32),
            pltpu.VMEM((CPAD,), jnp.float32),
            pltpu.VMEM((IPAD,), jnp.float32),
            pltpu.VMEM((NPW, D_H), jnp.float32),
            pltpu.VMEM((U, D_H), jnp.float32),
            pltpu.VMEM((2, 32), jnp.int32),
            pltpu.SemaphoreType.DMA,
        ],
    )(f, nA, hA, hB, nB, starts, coeff_pad, invdt_pad)
    return y


def _proj_body(x_ref, w_ref, b_ref, o_ref):
    o_ref[...] = (jnp.dot(x_ref[...], w_ref[...],
                          preferred_element_type=jnp.float32) + b_ref[...])


def _proj_res_body(y_ref, r_ref, w_ref, b_ref, x_ref, f_ref):
    x = y_ref[...] + r_ref[...]
    x_ref[...] = x
    f_ref[...] = (jnp.dot(x, w_ref[...],
                          preferred_element_type=jnp.float32) + b_ref[...])


def _proj(x, w, b):
    return pl.pallas_call(
        _proj_body,
        out_shape=jax.ShapeDtypeStruct((N, w.shape[1]), jnp.float32),
    )(x, w, b.reshape(1, -1))


def _proj_res(y, r, w, b):
    return pl.pallas_call(
        _proj_res_body,
        out_shape=(jax.ShapeDtypeStruct((N, D_H), jnp.float32),
                   jax.ShapeDtypeStruct((N, w.shape[1]), jnp.float32)),
    )(y, r, w, b.reshape(1, -1))


def kernel(node_features, node_idx, hedge_idx, W1, b1, W2, b2, W3, b3,
           W_out, b_out):
    nnz = node_idx.shape[0]
    permA = jnp.argsort(hedge_idx)
    hA = hedge_idx[permA]
    nA = node_idx[permA]
    permB = jnp.argsort(node_idx)
    nB = node_idx[permB]
    hB = hedge_idx[permB]

    bh = jnp.searchsorted(hA, jnp.arange(E + 1, dtype=jnp.int32)).astype(jnp.int32)
    De_i = jnp.maximum(bh[1:] - bh[:-1], 1)
    bn = jnp.searchsorted(nB, jnp.arange(N + 1, dtype=jnp.int32)).astype(jnp.int32)
    D1_i = bn[1:] - bn[:-1]
    cw = jnp.concatenate([jnp.zeros((1,), jnp.int32), jnp.cumsum(De_i[hB])])
    D0_i = cw[bn[1:]] - cw[bn[:-1]]
    De = De_i.astype(jnp.float32)
    Dt = (L0 * D0_i.astype(jnp.float32) + L1 * D1_i.astype(jnp.float32) + 1.0)
    invdt = 1.0 / Dt
    coeff = L0 + L1 / De

    sA = jnp.searchsorted(hA, jnp.arange(NW + 1, dtype=jnp.int32) * EPW).astype(jnp.int32)
    sB = jnp.searchsorted(nB, jnp.arange(NW + 1, dtype=jnp.int32) * NPW).astype(jnp.int32)
    starts = jnp.zeros((2, 32), jnp.int32)
    starts = starts.at[0, :NW + 1].set(sA).at[1, :NW + 1].set(sB)

    pad = 2 * G
    zpad = jnp.zeros((pad,), jnp.int32)
    nA_p = jnp.concatenate([nA, zpad])
    hA_p = jnp.concatenate([hA, zpad])
    hB_p = jnp.concatenate([hB, zpad])
    nB_p = jnp.concatenate([nB, zpad])

    coeff_pad = jnp.pad(
        jnp.concatenate([coeff, jnp.ones((EPAD - E,), jnp.float32)])
        .reshape(NW, EPW),
        ((0, 0), (0, CPAD - EPW)), constant_values=1.0)
    invdt_pad = jnp.pad(
        jnp.concatenate([invdt, jnp.ones((NPAD - N,), jnp.float32)])
        .reshape(NW, NPW),
        ((0, 0), (0, IPAD - NPW)), constant_values=1.0)

    def prop(f):
        fp = jnp.pad(f, ((0, NPAD - N), (0, 0)))
        y = _sc_layer(fp, nA_p, hA_p, hB_p, nB_p, starts, coeff_pad,
                      invdt_pad)
        return y[:N]

    f1 = _proj(node_features, W1, b1)
    Y1 = prop(f1)
    f2 = _proj(Y1, W2, b2)
    Y2 = prop(f2)
    x3, f3 = _proj_res(Y2, Y1, W3, b3)
    Y3 = prop(f3)
    _, out = _proj_res(Y3, x3, W_out, b_out)
    return out
